# baseline (speedup 1.0000x reference)
import functools

import jax
import jax.numpy as jnp
from jax import lax
from jax.experimental import pallas as pl
from jax.experimental.pallas import tpu as pltpu
from jax.experimental.pallas import tpu_sc as plsc

N = 10000
E = 320000
D = 128
ED = 16
A = 32
B = 8

NC = 2
NS = 16
NW = NC * NS
C = 64
NCH = 160
EPW = NCH * C
EP = NW * EPW
N2 = N + 8
RPT8 = 632
RLAST = N2 - (NS - 1) * RPT8
EBLK = 5120
NSLOT = 4


def _edge_mlp(edge_attr, w1, b1, w2, b2):
    def body(a_ref, w1_ref, b1_ref, w2_ref, b2_ref, e1_ref, e2_ref):
        a = a_ref[...]
        e1_ref[...] = (
            jnp.dot(a, w1_ref[...], preferred_element_type=jnp.float32) + b1_ref[...]
        )
        e2_ref[...] = (
            jnp.dot(a, w2_ref[...], preferred_element_type=jnp.float32) + b2_ref[...]
        )

    return pl.pallas_call(
        body,
        grid=(EP // EBLK,),
        in_specs=[
            pl.BlockSpec((EBLK, ED), lambda i: (i, 0)),
            pl.BlockSpec((ED, D), lambda i: (0, 0)),
            pl.BlockSpec((1, D), lambda i: (0, 0)),
            pl.BlockSpec((ED, D), lambda i: (0, 0)),
            pl.BlockSpec((1, D), lambda i: (0, 0)),
        ],
        out_specs=[
            pl.BlockSpec((EBLK, D), lambda i: (i, 0)),
            pl.BlockSpec((EBLK, D), lambda i: (i, 0)),
        ],
        out_shape=[
            jax.ShapeDtypeStruct((EP, D), jnp.float32),
            jax.ShapeDtypeStruct((EP, D), jnp.float32),
        ],
    )(edge_attr, w1, b1.reshape(1, D), w2, b2.reshape(1, D))


def _sc_conv(x, e, srcp, dstp):
    mesh = plsc.VectorSubcoreMesh(core_axis_name="c", subcore_axis_name="s")

    scratch = (
        [pltpu.VMEM((C, D), jnp.float32) for _ in range(NSLOT)]
        + [pltpu.VMEM((C,), jnp.int32) for _ in range(NSLOT)]
        + [pltpu.VMEM((C,), jnp.int32) for _ in range(NSLOT)]
        + [pltpu.SemaphoreType.DMA for _ in range(3 * NSLOT)]
        + [pltpu.VMEM_SHARED((N2, D), jnp.float32)]
    )

    @functools.partial(
        pl.kernel,
        out_type=jax.ShapeDtypeStruct((NC, N2, D), jnp.float32),
        mesh=mesh,
        scratch_types=scratch,
    )
    def conv(x_hbm, e_hbm, src_hbm, dst_hbm, out_hbm, *refs):
        ebufs = refs[0:NSLOT]
        srcbs = refs[NSLOT:2 * NSLOT]
        dstbs = refs[2 * NSLOT:3 * NSLOT]
        Ls = refs[3 * NSLOT:4 * NSLOT]
        Gs = refs[4 * NSLOT:5 * NSLOT]
        Ss = refs[5 * NSLOT:6 * NSLOT]
        accum = refs[6 * NSLOT]

        cid = lax.axis_index("c")
        sid = lax.axis_index("s")
        wid = cid * NS + sid
        ebase = wid * EPW

        def zbody(r, _):
            for cc in range(D // 16):
                ebufs[0][r, pl.ds(cc * 16, 16)] = jnp.zeros((16,), jnp.float32)
            return 0

        lax.fori_loop(0, C, zbody, 0)

        @pl.when(sid < NS - 1)
        def _zero_main():
            r0 = sid * RPT8
            for k in range(RPT8 // C):
                pltpu.sync_copy(ebufs[0], accum.at[pl.ds(r0 + k * C, C)])
            rem = RPT8 - (RPT8 // C) * C
            pltpu.sync_copy(
                ebufs[0].at[pl.ds(0, rem)],
                accum.at[pl.ds(r0 + (RPT8 // C) * C, rem)],
            )

        @pl.when(sid == NS - 1)
        def _zero_last():
            r0 = (NS - 1) * RPT8
            for k in range(RLAST // C):
                pltpu.sync_copy(ebufs[0], accum.at[pl.ds(r0 + k * C, C)])
            rem = RLAST - (RLAST // C) * C
            pltpu.sync_copy(
                ebufs[0].at[pl.ds(0, rem)],
                accum.at[pl.ds(r0 + (RLAST // C) * C, rem)],
            )

        plsc.subcore_barrier()

        def issue_loads(jj, s):
            pltpu.async_copy(src_hbm.at[wid, jj], srcbs[s], Ls[s])
            pltpu.async_copy(dst_hbm.at[wid, jj], dstbs[s], Ls[s])
            pltpu.async_copy(e_hbm.at[pl.ds(ebase + jj * C, C)], ebufs[s], Ls[s])

        def wait_loads(s):
            pltpu.make_async_copy(src_hbm.at[wid, 0], srcbs[s], Ls[s]).wait()
            pltpu.make_async_copy(dst_hbm.at[wid, 0], dstbs[s], Ls[s]).wait()
            pltpu.make_async_copy(e_hbm.at[pl.ds(0, C)], ebufs[s], Ls[s]).wait()

        def issue_gather(s):
            pltpu.async_copy(x_hbm.at[srcbs[s]], ebufs[s], Gs[s], add=True)

        def wait_gather(s):
            pltpu.make_async_copy(e_hbm.at[pl.ds(0, C)], ebufs[s], Gs[s]).wait()

        def issue_scatter(s):
            pltpu.async_copy(ebufs[s], accum.at[dstbs[s]], Ss[s], add=True)

        def wait_scatter(s):
            pltpu.make_async_copy(e_hbm.at[pl.ds(0, C)], ebufs[s], Ss[s]).wait()

        def relu(s):
            def rbody(r, _):
                for cc in range(D // 16):
                    sl = pl.ds(cc * 16, 16)
                    ebufs[s][r, sl] = jnp.maximum(ebufs[s][r, sl], 0.0)
                return 0

            lax.fori_loop(0, C, rbody, 0)

        for s in range(3):
            issue_loads(s, s)
        wait_loads(0)
        issue_gather(0)

        def group(b, _):
            for u in range(NSLOT):
                j = b * NSLOT + u
                s = u
                s1 = (u + 1) % NSLOT
                s3 = (u + 3) % NSLOT

                @pl.when(j + 1 < NCH)
                def _start_gather():
                    wait_loads(s1)
                    issue_gather(s1)

                wait_gather(s)
                relu(s)
                issue_scatter(s)

                @pl.when(j >= 1)
                def _drain_scatter():
                    wait_scatter(s3)

                @pl.when(j + 3 < NCH)
                def _prefetch():
                    issue_loads(j + 3, s3)

            return 0

        lax.fori_loop(0, NCH // NSLOT, group, 0)
        wait_scatter((NCH - 1) % NSLOT)

        plsc.subcore_barrier()

        @pl.when(sid < NS - 1)
        def _dump_main():
            r0 = sid * RPT8
            pltpu.sync_copy(
                accum.at[pl.ds(r0, RPT8)], out_hbm.at[cid, pl.ds(r0, RPT8)]
            )

        @pl.when(sid == NS - 1)
        def _dump_last():
            r0 = (NS - 1) * RPT8
            pltpu.sync_copy(
                accum.at[pl.ds(r0, RLAST)], out_hbm.at[cid, pl.ds(r0, RLAST)]
            )

    return conv(x, e, srcp, dstp)


def _node_mlp(x, p, w, b, g, be):
    def body(x_ref, p_ref, w_ref, b_ref, g_ref, be_ref, o_ref):
        agg = p_ref[0, 0:N, :] + p_ref[1, 0:N, :]
        t = x_ref[...] + agg
        z = jnp.dot(t, w_ref[...], preferred_element_type=jnp.float32) + b_ref[...]
        mu = jnp.mean(z, axis=0, keepdims=True)
        var = jnp.mean((z - mu) ** 2, axis=0, keepdims=True)
        zn = (z - mu) / jnp.sqrt(var + 1e-5) * g_ref[...] + be_ref[...]
        o_ref[...] = jnp.maximum(zn, 0.0)

    return pl.pallas_call(
        body,
        out_shape=jax.ShapeDtypeStruct((N, D), jnp.float32),
    )(x, p, w, b.reshape(1, D), g.reshape(1, D), be.reshape(1, D))


def _tail(h, p2, w2, b2, g2, be2, wa1, ba1, ga1, bea1, wa2, ba2, ga2, bea2,
          wa3, ba3):
    def bn(u, gg, bb):
        m = jnp.mean(u, axis=0, keepdims=True)
        v = jnp.mean((u - m) ** 2, axis=0, keepdims=True)
        return (u - m) / jnp.sqrt(v + 1e-5) * gg + bb

    def body(h_ref, p_ref, w2_ref, b2_ref, g2_ref, be2_ref, wa1_ref, ba1_ref,
             ga1_ref, bea1_ref, wa2_ref, ba2_ref, ga2_ref, bea2_ref, wa3_ref,
             ba3_ref, o_ref):
        t = h_ref[...] + p_ref[0, 0:N, :] + p_ref[1, 0:N, :]
        z = jnp.dot(t, w2_ref[...], preferred_element_type=jnp.float32) + b2_ref[...]
        zn = bn(z, g2_ref[...], be2_ref[...])
        h2 = jax.nn.sigmoid(jnp.maximum(zn, 0.0))
        pooled = jnp.mean(h2.reshape(B, N // B, D), axis=1)
        a = jnp.maximum(
            bn(jnp.dot(pooled, wa1_ref[...], preferred_element_type=jnp.float32)
               + ba1_ref[...], ga1_ref[...], bea1_ref[...]), 0.0)
        a = jnp.maximum(
            bn(jnp.dot(a, wa2_ref[...], preferred_element_type=jnp.float32)
               + ba2_ref[...], ga2_ref[...], bea2_ref[...]), 0.0)
        o_ref[...] = jax.nn.sigmoid(
            jnp.dot(a, wa3_ref[...], preferred_element_type=jnp.float32)
            + ba3_ref[...])

    return pl.pallas_call(
        body,
        out_shape=jax.ShapeDtypeStruct((B, A), jnp.float32),
    )(h, p2, w2, b2.reshape(1, D), g2.reshape(1, D), be2.reshape(1, D),
      wa1, ba1.reshape(1, D), ga1.reshape(1, D), bea1.reshape(1, D),
      wa2, ba2.reshape(1, D), ga2.reshape(1, D), bea2.reshape(1, D),
      wa3, ba3.reshape(1, A))


def kernel(x, edge_index, edge_attr, batch, lin_e1_W, lin_e1_b, W1, b1, g1, be1,
           lin_e2_W, lin_e2_b, W2, b2, g2, be2, Wa1, ba1, ga1, bea1, Wa2, ba2,
           ga2, bea2, Wa3, ba3):
    del batch
    pad = EP - E
    srcp = jnp.concatenate(
        [edge_index[0].astype(jnp.int32), jnp.zeros((pad,), jnp.int32)]
    ).reshape(NW, NCH, C)
    dstp = jnp.concatenate(
        [edge_index[1].astype(jnp.int32), jnp.full((pad,), N, jnp.int32)]
    ).reshape(NW, NCH, C)
    attr_p = jnp.concatenate(
        [edge_attr, jnp.zeros((pad, ED), jnp.float32)], axis=0
    )
    e1, e2 = _edge_mlp(attr_p, lin_e1_W, lin_e1_b, lin_e2_W, lin_e2_b)
    p1 = _sc_conv(x, e1, srcp, dstp)
    h = _node_mlp(x, p1, W1, b1, g1, be1)
    p2 = _sc_conv(h, e2, srcp, dstp)
    return _tail(h, p2, W2, b2, g2, be2, Wa1, ba1, ga1, bea1, Wa2, ba2, ga2,
                 bea2, Wa3, ba3)


# speedup vs baseline: 1.1086x; 1.1086x over previous
import functools

import jax
import jax.numpy as jnp
from jax import lax
from jax.experimental import pallas as pl
from jax.experimental.pallas import tpu as pltpu
from jax.experimental.pallas import tpu_sc as plsc

N = 10000
E = 320000
D = 128
ED = 16
A = 32
B = 8

NC = 2
NS = 16
NW = NC * NS
C = 128
NCH = 79
EPW = NCH * C
EP = NW * EPW
N2 = N + 8
RPT8 = 632
RLAST = N2 - (NS - 1) * RPT8
EBLK = 5056


def _edge_mlp(edge_attr, w1, b1, w2, b2):
    def body(a_ref, w1_ref, b1_ref, w2_ref, b2_ref, e1_ref, e2_ref):
        a = a_ref[...]
        e1_ref[...] = (
            jnp.dot(a, w1_ref[...], preferred_element_type=jnp.float32) + b1_ref[...]
        )
        e2_ref[...] = (
            jnp.dot(a, w2_ref[...], preferred_element_type=jnp.float32) + b2_ref[...]
        )

    return pl.pallas_call(
        body,
        grid=(EP // EBLK,),
        in_specs=[
            pl.BlockSpec((EBLK, ED), lambda i: (i, 0)),
            pl.BlockSpec((ED, D), lambda i: (0, 0)),
            pl.BlockSpec((1, D), lambda i: (0, 0)),
            pl.BlockSpec((ED, D), lambda i: (0, 0)),
            pl.BlockSpec((1, D), lambda i: (0, 0)),
        ],
        out_specs=[
            pl.BlockSpec((EBLK, D), lambda i: (i, 0)),
            pl.BlockSpec((EBLK, D), lambda i: (i, 0)),
        ],
        out_shape=[
            jax.ShapeDtypeStruct((EP, D), jnp.float32),
            jax.ShapeDtypeStruct((EP, D), jnp.float32),
        ],
    )(edge_attr, w1, b1.reshape(1, D), w2, b2.reshape(1, D))


def _sc_conv(x, e, srcp, dstp):
    mesh = plsc.VectorSubcoreMesh(core_axis_name="c", subcore_axis_name="s")

    @functools.partial(
        pl.kernel,
        out_type=jax.ShapeDtypeStruct((NC, N2, D), jnp.float32),
        mesh=mesh,
        scratch_types=[
            pltpu.VMEM((NCH, C), jnp.int32),
            pltpu.VMEM((NCH, C), jnp.int32),
            pltpu.VMEM((C, D), jnp.float32),
            pltpu.VMEM_SHARED((N2, D), jnp.float32),
        ],
    )
    def conv(x_hbm, e_hbm, src_hbm, dst_hbm, out_hbm, srcb, dstb, ebuf, accum):
        cid = lax.axis_index("c")
        sid = lax.axis_index("s")
        wid = cid * NS + sid

        def zbody(r, _):
            for cc in range(D // 16):
                ebuf[r, pl.ds(cc * 16, 16)] = jnp.zeros((16,), jnp.float32)
            return 0

        lax.fori_loop(0, C, zbody, 0)

        @pl.when(sid < NS - 1)
        def _zero_main():
            r0 = sid * RPT8
            for k in range(RPT8 // C):
                pltpu.sync_copy(ebuf, accum.at[pl.ds(r0 + k * C, C)])
            rem = RPT8 - (RPT8 // C) * C
            pltpu.sync_copy(
                ebuf.at[pl.ds(0, rem)],
                accum.at[pl.ds(r0 + (RPT8 // C) * C, rem)],
            )

        @pl.when(sid == NS - 1)
        def _zero_last():
            r0 = (NS - 1) * RPT8
            for k in range(RLAST // C):
                pltpu.sync_copy(ebuf, accum.at[pl.ds(r0 + k * C, C)])
            rem = RLAST - (RLAST // C) * C
            pltpu.sync_copy(
                ebuf.at[pl.ds(0, rem)],
                accum.at[pl.ds(r0 + (RLAST // C) * C, rem)],
            )

        pltpu.sync_copy(src_hbm.at[wid], srcb)
        pltpu.sync_copy(dst_hbm.at[wid], dstb)
        plsc.subcore_barrier()

        ebase = wid * EPW

        def chunk_body(g, _):
            pltpu.sync_copy(e_hbm.at[pl.ds(ebase + g * C, C)], ebuf)
            pltpu.sync_copy(x_hbm.at[srcb.at[g]], ebuf, add=True)

            def rbody(r, _):
                for cc in range(D // 16):
                    sl = pl.ds(cc * 16, 16)
                    ebuf[r, sl] = jnp.maximum(ebuf[r, sl], 0.0)
                return 0

            lax.fori_loop(0, C, rbody, 0)
            pltpu.sync_copy(ebuf, accum.at[dstb.at[g]], add=True)
            return 0

        lax.fori_loop(0, NCH, chunk_body, 0)

        plsc.subcore_barrier()

        @pl.when(sid < NS - 1)
        def _dump_main():
            r0 = sid * RPT8
            pltpu.sync_copy(
                accum.at[pl.ds(r0, RPT8)], out_hbm.at[cid, pl.ds(r0, RPT8)]
            )

        @pl.when(sid == NS - 1)
        def _dump_last():
            r0 = (NS - 1) * RPT8
            pltpu.sync_copy(
                accum.at[pl.ds(r0, RLAST)], out_hbm.at[cid, pl.ds(r0, RLAST)]
            )

    return conv(x, e, srcp, dstp)


def _node_mlp(x, p, w, b, g, be):
    def body(x_ref, p_ref, w_ref, b_ref, g_ref, be_ref, o_ref):
        agg = p_ref[0, 0:N, :] + p_ref[1, 0:N, :]
        t = x_ref[...] + agg
        z = jnp.dot(t, w_ref[...], preferred_element_type=jnp.float32) + b_ref[...]
        mu = jnp.mean(z, axis=0, keepdims=True)
        var = jnp.mean((z - mu) ** 2, axis=0, keepdims=True)
        zn = (z - mu) / jnp.sqrt(var + 1e-5) * g_ref[...] + be_ref[...]
        o_ref[...] = jnp.maximum(zn, 0.0)

    return pl.pallas_call(
        body,
        out_shape=jax.ShapeDtypeStruct((N, D), jnp.float32),
    )(x, p, w, b.reshape(1, D), g.reshape(1, D), be.reshape(1, D))


def _tail(h, p2, w2, b2, g2, be2, wa1, ba1, ga1, bea1, wa2, ba2, ga2, bea2,
          wa3, ba3):
    def bn(u, gg, bb):
        m = jnp.mean(u, axis=0, keepdims=True)
        v = jnp.mean((u - m) ** 2, axis=0, keepdims=True)
        return (u - m) / jnp.sqrt(v + 1e-5) * gg + bb

    def body(h_ref, p_ref, w2_ref, b2_ref, g2_ref, be2_ref, wa1_ref, ba1_ref,
             ga1_ref, bea1_ref, wa2_ref, ba2_ref, ga2_ref, bea2_ref, wa3_ref,
             ba3_ref, o_ref):
        t = h_ref[...] + p_ref[0, 0:N, :] + p_ref[1, 0:N, :]
        z = jnp.dot(t, w2_ref[...], preferred_element_type=jnp.float32) + b2_ref[...]
        zn = bn(z, g2_ref[...], be2_ref[...])
        h2 = jax.nn.sigmoid(jnp.maximum(zn, 0.0))
        pooled = jnp.mean(h2.reshape(B, N // B, D), axis=1)
        a = jnp.maximum(
            bn(jnp.dot(pooled, wa1_ref[...], preferred_element_type=jnp.float32)
               + ba1_ref[...], ga1_ref[...], bea1_ref[...]), 0.0)
        a = jnp.maximum(
            bn(jnp.dot(a, wa2_ref[...], preferred_element_type=jnp.float32)
               + ba2_ref[...], ga2_ref[...], bea2_ref[...]), 0.0)
        o_ref[...] = jax.nn.sigmoid(
            jnp.dot(a, wa3_ref[...], preferred_element_type=jnp.float32)
            + ba3_ref[...])

    return pl.pallas_call(
        body,
        out_shape=jax.ShapeDtypeStruct((B, A), jnp.float32),
    )(h, p2, w2, b2.reshape(1, D), g2.reshape(1, D), be2.reshape(1, D),
      wa1, ba1.reshape(1, D), ga1.reshape(1, D), bea1.reshape(1, D),
      wa2, ba2.reshape(1, D), ga2.reshape(1, D), bea2.reshape(1, D),
      wa3, ba3.reshape(1, A))


def kernel(x, edge_index, edge_attr, batch, lin_e1_W, lin_e1_b, W1, b1, g1, be1,
           lin_e2_W, lin_e2_b, W2, b2, g2, be2, Wa1, ba1, ga1, bea1, Wa2, ba2,
           ga2, bea2, Wa3, ba3):
    del batch
    pad = EP - E
    srcp = jnp.concatenate(
        [edge_index[0].astype(jnp.int32), jnp.zeros((pad,), jnp.int32)]
    ).reshape(NW, NCH, C)
    dstp = jnp.concatenate(
        [edge_index[1].astype(jnp.int32), jnp.full((pad,), N, jnp.int32)]
    ).reshape(NW, NCH, C)
    attr_p = jnp.concatenate(
        [edge_attr, jnp.zeros((pad, ED), jnp.float32)], axis=0
    )
    e1, e2 = _edge_mlp(attr_p, lin_e1_W, lin_e1_b, lin_e2_W, lin_e2_b)
    p1 = _sc_conv(x, e1, srcp, dstp)
    h = _node_mlp(x, p1, W1, b1, g1, be1)
    p2 = _sc_conv(h, e2, srcp, dstp)
    return _tail(h, p2, W2, b2, g2, be2, Wa1, ba1, ga1, bea1, Wa2, ba2, ga2,
                 bea2, Wa3, ba3)


# speedup vs baseline: 1.5060x; 1.3584x over previous
import functools

import jax
import jax.numpy as jnp
from jax import lax
from jax.experimental import pallas as pl
from jax.experimental.pallas import tpu as pltpu
from jax.experimental.pallas import tpu_sc as plsc

N = 10000
E = 320000
D = 128
ED = 16
A = 32
B = 8

NC = 2
NS = 16
NW = NC * NS
C = 128
NCH = 79
EPW = NCH * C
EP = NW * EPW
N2 = N + 8
RPT8 = 632
RLAST = N2 - (NS - 1) * RPT8
EBLK = 5056


def _edge_mlp(edge_attr, w1, b1, w2, b2):
    def body(a_ref, w1_ref, b1_ref, w2_ref, b2_ref, e1_ref, e2_ref):
        a = a_ref[...]
        e1_ref[...] = (
            jnp.dot(a, w1_ref[...], preferred_element_type=jnp.float32) + b1_ref[...]
        )
        e2_ref[...] = (
            jnp.dot(a, w2_ref[...], preferred_element_type=jnp.float32) + b2_ref[...]
        )

    return pl.pallas_call(
        body,
        grid=(EP // EBLK,),
        in_specs=[
            pl.BlockSpec((EBLK, ED), lambda i: (i, 0)),
            pl.BlockSpec((ED, D), lambda i: (0, 0)),
            pl.BlockSpec((1, D), lambda i: (0, 0)),
            pl.BlockSpec((ED, D), lambda i: (0, 0)),
            pl.BlockSpec((1, D), lambda i: (0, 0)),
        ],
        out_specs=[
            pl.BlockSpec((EBLK, D), lambda i: (i, 0)),
            pl.BlockSpec((EBLK, D), lambda i: (i, 0)),
        ],
        out_shape=[
            jax.ShapeDtypeStruct((EP, D), jnp.float32),
            jax.ShapeDtypeStruct((EP, D), jnp.float32),
        ],
    )(edge_attr, w1, b1.reshape(1, D), w2, b2.reshape(1, D))


def _sc_conv(x, e, srcp, dstp):
    mesh = plsc.VectorSubcoreMesh(core_axis_name="c", subcore_axis_name="s")

    @functools.partial(
        pl.kernel,
        out_type=jax.ShapeDtypeStruct((NC, N2, D), jnp.float32),
        mesh=mesh,
        scratch_types=[
            pltpu.VMEM((NCH, C), jnp.int32),
            pltpu.VMEM((NCH, C), jnp.int32),
            pltpu.VMEM((C, D), jnp.float32),
            pltpu.VMEM_SHARED((N2, D), jnp.float32),
        ],
    )
    def conv(x_hbm, e_hbm, src_hbm, dst_hbm, out_hbm, srcb, dstb, ebuf, accum):
        cid = lax.axis_index("c")
        sid = lax.axis_index("s")
        wid = cid * NS + sid

        def zbody(r, _):
            for cc in range(D // 16):
                ebuf[r, pl.ds(cc * 16, 16)] = jnp.zeros((16,), jnp.float32)
            return 0

        lax.fori_loop(0, C, zbody, 0)

        @pl.when(sid < NS - 1)
        def _zero_main():
            r0 = sid * RPT8
            for k in range(RPT8 // C):
                pltpu.sync_copy(ebuf, accum.at[pl.ds(r0 + k * C, C)])
            rem = RPT8 - (RPT8 // C) * C
            pltpu.sync_copy(
                ebuf.at[pl.ds(0, rem)],
                accum.at[pl.ds(r0 + (RPT8 // C) * C, rem)],
            )

        @pl.when(sid == NS - 1)
        def _zero_last():
            r0 = (NS - 1) * RPT8
            for k in range(RLAST // C):
                pltpu.sync_copy(ebuf, accum.at[pl.ds(r0 + k * C, C)])
            rem = RLAST - (RLAST // C) * C
            pltpu.sync_copy(
                ebuf.at[pl.ds(0, rem)],
                accum.at[pl.ds(r0 + (RLAST // C) * C, rem)],
            )

        pltpu.sync_copy(src_hbm.at[wid], srcb)
        pltpu.sync_copy(dst_hbm.at[wid], dstb)
        plsc.subcore_barrier()

        ebase = wid * EPW

        def chunk_body(g, _):
            pltpu.sync_copy(e_hbm.at[pl.ds(ebase + g * C, C)], ebuf)
            pltpu.sync_copy(x_hbm.at[srcb.at[g]], ebuf, add=True)

            def rbody(r, _):
                for cc in range(D // 16):
                    sl = pl.ds(cc * 16, 16)
                    ebuf[r, sl] = jnp.maximum(ebuf[r, sl], 0.0)
                return 0

            lax.fori_loop(0, C, rbody, 0)
            pltpu.sync_copy(ebuf, accum.at[dstb.at[g]], add=True)
            return 0

        lax.fori_loop(0, NCH, chunk_body, 0)

        plsc.subcore_barrier()

        @pl.when(sid < NS - 1)
        def _dump_main():
            r0 = sid * RPT8
            pltpu.sync_copy(
                accum.at[pl.ds(r0, RPT8)], out_hbm.at[cid, pl.ds(r0, RPT8)]
            )

        @pl.when(sid == NS - 1)
        def _dump_last():
            r0 = (NS - 1) * RPT8
            pltpu.sync_copy(
                accum.at[pl.ds(r0, RLAST)], out_hbm.at[cid, pl.ds(r0, RLAST)]
            )

    return conv(x, e, srcp, dstp)


def _node_mlp(x, p, w, b, g, be):
    def body(x_ref, p_ref, w_ref, b_ref, g_ref, be_ref, o_ref):
        agg = p_ref[0, 0:N, :] + p_ref[1, 0:N, :]
        t = x_ref[...] + agg
        z = jnp.dot(t, w_ref[...], preferred_element_type=jnp.float32) + b_ref[...]
        mu = jnp.mean(z, axis=0, keepdims=True)
        var = jnp.mean((z - mu) ** 2, axis=0, keepdims=True)
        zn = (z - mu) / jnp.sqrt(var + 1e-5) * g_ref[...] + be_ref[...]
        o_ref[...] = jnp.maximum(zn, 0.0)

    return pl.pallas_call(
        body,
        out_shape=jax.ShapeDtypeStruct((N, D), jnp.float32),
    )(x, p, w, b.reshape(1, D), g.reshape(1, D), be.reshape(1, D))


def _tail(h, p2, w2, b2, g2, be2, wa1, ba1, ga1, bea1, wa2, ba2, ga2, bea2,
          wa3, ba3):
    def bn(u, gg, bb):
        m = jnp.mean(u, axis=0, keepdims=True)
        v = jnp.mean((u - m) ** 2, axis=0, keepdims=True)
        return (u - m) / jnp.sqrt(v + 1e-5) * gg + bb

    def body(h_ref, p_ref, w2_ref, b2_ref, g2_ref, be2_ref, wa1_ref, ba1_ref,
             ga1_ref, bea1_ref, wa2_ref, ba2_ref, ga2_ref, bea2_ref, wa3_ref,
             ba3_ref, o_ref):
        t = h_ref[...] + p_ref[0, 0:N, :] + p_ref[1, 0:N, :]
        z = jnp.dot(t, w2_ref[...], preferred_element_type=jnp.float32) + b2_ref[...]
        zn = bn(z, g2_ref[...], be2_ref[...])
        h2 = jax.nn.sigmoid(jnp.maximum(zn, 0.0))
        pooled = jnp.mean(h2.reshape(B, N // B, D), axis=1)
        a = jnp.maximum(
            bn(jnp.dot(pooled, wa1_ref[...], preferred_element_type=jnp.float32)
               + ba1_ref[...], ga1_ref[...], bea1_ref[...]), 0.0)
        a = jnp.maximum(
            bn(jnp.dot(a, wa2_ref[...], preferred_element_type=jnp.float32)
               + ba2_ref[...], ga2_ref[...], bea2_ref[...]), 0.0)
        o_ref[...] = jax.nn.sigmoid(
            jnp.dot(a, wa3_ref[...], preferred_element_type=jnp.float32)
            + ba3_ref[...])

    return pl.pallas_call(
        body,
        out_shape=jax.ShapeDtypeStruct((B, A), jnp.float32),
    )(h, p2, w2, b2.reshape(1, D), g2.reshape(1, D), be2.reshape(1, D),
      wa1, ba1.reshape(1, D), ga1.reshape(1, D), bea1.reshape(1, D),
      wa2, ba2.reshape(1, D), ga2.reshape(1, D), bea2.reshape(1, D),
      wa3, ba3.reshape(1, A))


def kernel(x, edge_index, edge_attr, batch, lin_e1_W, lin_e1_b, W1, b1, g1, be1,
           lin_e2_W, lin_e2_b, W2, b2, g2, be2, Wa1, ba1, ga1, bea1, Wa2, ba2,
           ga2, bea2, Wa3, ba3):
    del batch
    pad = EP - E
    pad_src = jnp.arange(pad, dtype=jnp.int32) % N
    pad_dst = N + (jnp.arange(pad, dtype=jnp.int32) % (N2 - N))
    srcp = jnp.concatenate(
        [edge_index[0].astype(jnp.int32), pad_src]
    ).reshape(NW, NCH, C)
    dstp = jnp.concatenate(
        [edge_index[1].astype(jnp.int32), pad_dst]
    ).reshape(NW, NCH, C)
    attr_p = jnp.concatenate(
        [edge_attr, jnp.zeros((pad, ED), jnp.float32)], axis=0
    )
    e1, e2 = _edge_mlp(attr_p, lin_e1_W, lin_e1_b, lin_e2_W, lin_e2_b)
    p1 = _sc_conv(x, e1, srcp, dstp)
    h = _node_mlp(x, p1, W1, b1, g1, be1)
    p2 = _sc_conv(h, e2, srcp, dstp)
    return _tail(h, p2, W2, b2, g2, be2, Wa1, ba1, ga1, bea1, Wa2, ba2, ga2,
                 bea2, Wa3, ba3)
